# baseline (speedup 1.0000x reference)
import functools

import jax
import jax.numpy as jnp
from jax import lax
from jax.experimental import pallas as pl
from jax.experimental.pallas import tpu as pltpu
from jax.experimental.pallas import tpu_sc as plsc

B = 16
N = 384 * 384
BITS = 15
NB = 1 << BITS
SHIFT = 32 - BITS
HALF = 1 << (SHIFT - 1)
L = 16
CHUNK = 4096
NCHUNK = N // CHUNK
HCH = NCHUNK // 2
HIST = 2 * NB
NUP = (NB // 2) // L
MCH = 4096
UNROLL = 4


def _hist_chunk(lb, yb, hist, ys):
    ones = jnp.ones((L,), jnp.float32)

    def vb(i, ysacc):
        for u in range(UNROLL):
            off = (i * UNROLL + u) * L
            l = lb[pl.ds(off, L)]
            y = yb[pl.ds(off, L)]
            yf = y.astype(jnp.float32)
            e = (1.0 + l) - 2.0 * (l * yf)
            bits = plsc.bitcast(e, jnp.uint32)
            neg = bits >> 31
            m = (jnp.uint32(0) - neg) | jnp.uint32(0x80000000)
            key = bits ^ m
            ksat = jnp.minimum(key, jnp.uint32(0xFFFEFFFF))
            bkt = (ksat + jnp.uint32(HALF)) >> SHIFT
            yu = plsc.bitcast(y, jnp.uint32)
            idx = plsc.bitcast(bkt | (yu << BITS), jnp.int32)
            plsc.addupdate_scatter(hist, [idx], ones)
            ysacc = ysacc + yf
        return ysacc

    return lax.fori_loop(0, CHUNK // L // UNROLL, vb, ys)


def _make_kernel():
    mesh = plsc.VectorSubcoreMesh(
        core_axis_name="c", subcore_axis_name="s", num_cores=2,
        num_subcores=16)

    @functools.partial(
        pl.kernel,
        out_type=jax.ShapeDtypeStruct((B, L), jnp.float32),
        mesh=mesh,
        scratch_types=[
            pltpu.VMEM((CHUNK,), jnp.float32),
            pltpu.VMEM((CHUNK,), jnp.float32),
            pltpu.VMEM((CHUNK,), jnp.int32),
            pltpu.VMEM((CHUNK,), jnp.int32),
            pltpu.VMEM((HIST,), jnp.float32),
            pltpu.VMEM((MCH,), jnp.float32),
            pltpu.VMEM((L,), jnp.float32),
            pltpu.VMEM((L,), jnp.float32),
            pltpu.VMEM_SHARED((8, HIST), jnp.float32),
            pltpu.VMEM_SHARED((16, L), jnp.float32),
            pltpu.SemaphoreType.DMA,
            pltpu.SemaphoreType.DMA,
            pltpu.SemaphoreType.DMA,
            pltpu.SemaphoreType.DMA,
        ],
        compiler_params=pltpu.CompilerParams(needs_layout_passes=False),
    )
    def kern(logits_hbm, labels_hbm, out_hbm, lbuf0, lbuf1, ybuf0, ybuf1,
             hist, mbuf, ysp, obuf, sh_hist, sh_ys, sl0, sy0, sl1, sy1):
        c = lax.axis_index("c")
        s = lax.axis_index("s")
        slot = s % 8
        role = s // 8
        img = c * 8 + slot
        first = role * HCH

        zeros = jnp.zeros((L,), jnp.float32)

        with jax.named_scope("ph_zero"):
            def zero_body(i, _):
                for u in range(8):
                    hist[pl.ds((i * 8 + u) * L, L)] = zeros
                return 0

            lax.fori_loop(0, HIST // L // 8, zero_body, 0)

        def start(k, lb, yb, sl, sy):
            off = img * N + k * CHUNK
            pltpu.async_copy(logits_hbm.at[pl.ds(off, CHUNK)], lb, sl)
            pltpu.async_copy(labels_hbm.at[pl.ds(off, CHUNK)], yb, sy)

        def wait(lb, yb, sl, sy, k):
            off = img * N + k * CHUNK
            pltpu.make_async_copy(
                logits_hbm.at[pl.ds(off, CHUNK)], lb, sl).wait()
            pltpu.make_async_copy(
                labels_hbm.at[pl.ds(off, CHUNK)], yb, sy).wait()

        with jax.named_scope("ph_hist"):
            start(first, lbuf0, ybuf0, sl0, sy0)

            def body(jj, ys):
                k = first + jj * 2
                start(k + 1, lbuf1, ybuf1, sl1, sy1)
                wait(lbuf0, ybuf0, sl0, sy0, k)
                ys = _hist_chunk(lbuf0, ybuf0, hist, ys)

                @pl.when(jj < HCH // 2 - 1)
                def _():
                    start(k + 2, lbuf0, ybuf0, sl0, sy0)

                wait(lbuf1, ybuf1, sl1, sy1, k + 1)
                ys = _hist_chunk(lbuf1, ybuf1, hist, ys)
                return ys

            ys = lax.fori_loop(0, HCH // 2, body, zeros)

        with jax.named_scope("ph_sync"):
            obuf[...] = ys
            pltpu.sync_copy(obuf, sh_ys.at[s])

            @pl.when(role == 1)
            def _():
                pltpu.sync_copy(hist, sh_hist.at[slot])

            plsc.subcore_barrier()

        @pl.when(role == 0)
        def _():
            with jax.named_scope("ph_merge"):
                def mbody(kk, _):
                    pltpu.sync_copy(sh_hist.at[slot, pl.ds(kk * MCH, MCH)],
                                    mbuf)

                    def madd(i, _):
                        for u in range(4):
                            off = (i * 4 + u) * L
                            hoff = kk * MCH + off
                            hist[pl.ds(hoff, L)] = (hist[pl.ds(hoff, L)]
                                                    + mbuf[pl.ds(off, L)])
                        return 0

                    lax.fori_loop(0, MCH // L // 4, madd, 0)
                    return 0

                lax.fori_loop(0, HIST // MCH, mbody, 0)
                pltpu.sync_copy(sh_ys.at[s + 8], ysp)

            with jax.named_scope("ph_scan"):
                G = jnp.sum(ys) + jnp.sum(ysp[...])
                G2 = jnp.float32(N) - G

                iota = lax.iota(jnp.int32, L)
                one = jnp.full((L,), 1.0, jnp.float32)
                lane15 = jnp.full((L,), L - 1, jnp.int32)

                def scan_step(j, acc, cnv, cpv):
                    base = NB - (j + 1) * L
                    Pd = lax.rev(hist[pl.ds(NB + base, L)], (0,))
                    Qd = lax.rev(hist[pl.ds(base, L)], (0,))
                    ip = plsc.cumsum(Pd)
                    iq = plsc.cumsum(Qd)
                    pbar = cpv + (ip - Pd)
                    nbar = cnv + (iq - Qd)
                    d1 = G + nbar
                    num1 = (G - pbar) * Qd + Pd * d1
                    diff1 = jnp.where(d1 == 0.0, one,
                                      num1 / (d1 * (d1 + Qd)))
                    d2 = G2 + pbar
                    num2 = (G2 - nbar) * Pd + Qd * d2
                    diff2 = jnp.where(d2 == 0.0, one,
                                      num2 / (d2 * (d2 + Pd)))
                    hvec = (NB - 1 - j * L) - iota
                    center = plsc.bitcast(hvec, jnp.uint32) << SHIFT
                    eh = plsc.bitcast(center ^ jnp.uint32(0x80000000),
                                      jnp.float32)
                    relu = jnp.maximum(eh, 0.0)
                    contrib = relu * (diff1 + diff2)
                    cnt = Pd + Qd
                    acc = acc + jnp.where(cnt > 0.0, contrib, zeros)
                    cnv = cnv + jnp.take_along_axis(iq, lane15, axis=0)
                    cpv = cpv + jnp.take_along_axis(ip, lane15, axis=0)
                    return acc, cnv, cpv

                def scan_body(jj, carry):
                    acc, cnv, cpv = carry
                    acc, cnv, cpv = scan_step(jj * 2, acc, cnv, cpv)
                    acc, cnv, cpv = scan_step(jj * 2 + 1, acc, cnv, cpv)
                    return (acc, cnv, cpv)

                acc, _, _ = lax.fori_loop(
                    0, NUP // 2, scan_body, (zeros, zeros, zeros))
                loss = jnp.sum(acc) * 0.5
                obuf[...] = jnp.full((L,), loss)
                pltpu.sync_copy(obuf, out_hbm.at[img])

    return kern


_kern = _make_kernel()


def kernel(logits, labels):
    lf = logits.reshape(B * N)
    yf = labels.reshape(B * N)
    out = _kern(lf, yf)
    return jnp.mean(out[:, 0])


# speedup vs baseline: 1.0175x; 1.0175x over previous
import functools

import jax
import jax.numpy as jnp
from jax import lax
from jax.experimental import pallas as pl
from jax.experimental.pallas import tpu as pltpu
from jax.experimental.pallas import tpu_sc as plsc

B = 16
N = 384 * 384
BITS = 15
NB = 1 << BITS
SHIFT = 32 - BITS
HALF = 1 << (SHIFT - 1)
L = 16
CHUNK = 4096
NCHUNK = N // CHUNK
HCH = NCHUNK // 2
HIST = 2 * NB
NUP = (NB // 2) // L
MCH = 4096
UNROLL = 8


def _hist_chunk(lb, yb, hist, ys):
    ones = jnp.ones((L,), jnp.float32)

    def vb(i, ysacc):
        for u in range(UNROLL):
            off = (i * UNROLL + u) * L
            l = lb[pl.ds(off, L)]
            y = yb[pl.ds(off, L)]
            yf = y.astype(jnp.float32)
            e = (1.0 + l) - 2.0 * (l * yf)
            bi = plsc.bitcast(e, jnp.int32)
            m = plsc.bitcast(bi >> 31, jnp.uint32) | jnp.uint32(0x80000000)
            key = plsc.bitcast(bi, jnp.uint32) ^ m
            bkt = (key + jnp.uint32(HALF)) >> SHIFT
            yu = plsc.bitcast(y, jnp.uint32)
            idx = plsc.bitcast(bkt | (yu << BITS), jnp.int32)
            plsc.addupdate_scatter(hist, [idx], ones)
            ysacc = ysacc + yf
        return ysacc

    return lax.fori_loop(0, CHUNK // L // UNROLL, vb, ys)


def _make_kernel():
    mesh = plsc.VectorSubcoreMesh(
        core_axis_name="c", subcore_axis_name="s", num_cores=2,
        num_subcores=16)

    @functools.partial(
        pl.kernel,
        out_type=jax.ShapeDtypeStruct((B, L), jnp.float32),
        mesh=mesh,
        scratch_types=[
            pltpu.VMEM((CHUNK,), jnp.float32),
            pltpu.VMEM((CHUNK,), jnp.float32),
            pltpu.VMEM((CHUNK,), jnp.int32),
            pltpu.VMEM((CHUNK,), jnp.int32),
            pltpu.VMEM((HIST,), jnp.float32),
            pltpu.VMEM((MCH,), jnp.float32),
            pltpu.VMEM((L,), jnp.float32),
            pltpu.VMEM((L,), jnp.float32),
            pltpu.VMEM_SHARED((8, HIST), jnp.float32),
            pltpu.VMEM_SHARED((16, L), jnp.float32),
            pltpu.SemaphoreType.DMA,
            pltpu.SemaphoreType.DMA,
            pltpu.SemaphoreType.DMA,
            pltpu.SemaphoreType.DMA,
        ],
        compiler_params=pltpu.CompilerParams(needs_layout_passes=False),
    )
    def kern(logits_hbm, labels_hbm, out_hbm, lbuf0, lbuf1, ybuf0, ybuf1,
             hist, mbuf, ysp, obuf, sh_hist, sh_ys, sl0, sy0, sl1, sy1):
        c = lax.axis_index("c")
        s = lax.axis_index("s")
        slot = s % 8
        role = s // 8
        img = c * 8 + slot
        first = role * HCH

        zeros = jnp.zeros((L,), jnp.float32)

        def start(k, lb, yb, sl, sy):
            off = img * N + k * CHUNK
            pltpu.async_copy(logits_hbm.at[pl.ds(off, CHUNK)], lb, sl)
            pltpu.async_copy(labels_hbm.at[pl.ds(off, CHUNK)], yb, sy)

        def wait(lb, yb, sl, sy, k):
            off = img * N + k * CHUNK
            pltpu.make_async_copy(
                logits_hbm.at[pl.ds(off, CHUNK)], lb, sl).wait()
            pltpu.make_async_copy(
                labels_hbm.at[pl.ds(off, CHUNK)], yb, sy).wait()

        start(first, lbuf0, ybuf0, sl0, sy0)

        with jax.named_scope("ph_zero"):
            def zero_body(i, _):
                for u in range(8):
                    hist[pl.ds((i * 8 + u) * L, L)] = zeros
                return 0

            lax.fori_loop(0, HIST // L // 8, zero_body, 0)

        with jax.named_scope("ph_hist"):
            def body(jj, ys):
                k = first + jj * 2
                start(k + 1, lbuf1, ybuf1, sl1, sy1)
                wait(lbuf0, ybuf0, sl0, sy0, k)
                ys = _hist_chunk(lbuf0, ybuf0, hist, ys)

                @pl.when(jj < HCH // 2 - 1)
                def _():
                    start(k + 2, lbuf0, ybuf0, sl0, sy0)

                wait(lbuf1, ybuf1, sl1, sy1, k + 1)
                ys = _hist_chunk(lbuf1, ybuf1, hist, ys)
                return ys

            ys = lax.fori_loop(0, HCH // 2, body, zeros)

        with jax.named_scope("ph_sync"):
            obuf[...] = ys
            pltpu.sync_copy(obuf, sh_ys.at[s])

            @pl.when(role == 1)
            def _():
                pltpu.sync_copy(hist, sh_hist.at[slot])

            plsc.subcore_barrier()

        @pl.when(role == 0)
        def _():
            with jax.named_scope("ph_merge"):
                def mbody(kk, _):
                    pltpu.sync_copy(sh_hist.at[slot, pl.ds(kk * MCH, MCH)],
                                    mbuf)

                    def madd(i, _):
                        for u in range(4):
                            off = (i * 4 + u) * L
                            hoff = kk * MCH + off
                            hist[pl.ds(hoff, L)] = (hist[pl.ds(hoff, L)]
                                                    + mbuf[pl.ds(off, L)])
                        return 0

                    lax.fori_loop(0, MCH // L // 4, madd, 0)
                    return 0

                lax.fori_loop(0, HIST // MCH, mbody, 0)
                pltpu.sync_copy(sh_ys.at[s + 8], ysp)

            with jax.named_scope("ph_scan"):
                G = jnp.sum(ys) + jnp.sum(ysp[...])
                G2 = jnp.float32(N) - G

                iota = lax.iota(jnp.int32, L)
                one = jnp.full((L,), 1.0, jnp.float32)
                lane15 = jnp.full((L,), L - 1, jnp.int32)

                def scan_step(j, acc, cnv, cpv):
                    base = NB - (j + 1) * L
                    Pd = lax.rev(hist[pl.ds(NB + base, L)], (0,))
                    Qd = lax.rev(hist[pl.ds(base, L)], (0,))
                    ip = plsc.cumsum(Pd)
                    iq = plsc.cumsum(Qd)
                    pbar = cpv + (ip - Pd)
                    nbar = cnv + (iq - Qd)
                    d1 = G + nbar
                    num1 = (G - pbar) * Qd + Pd * d1
                    diff1 = jnp.where(d1 == 0.0, one,
                                      num1 / (d1 * (d1 + Qd)))
                    d2 = G2 + pbar
                    num2 = (G2 - nbar) * Pd + Qd * d2
                    diff2 = jnp.where(d2 == 0.0, one,
                                      num2 / (d2 * (d2 + Pd)))
                    hvec = (NB - 1 - j * L) - iota
                    center = plsc.bitcast(hvec, jnp.uint32) << SHIFT
                    eh = plsc.bitcast(center ^ jnp.uint32(0x80000000),
                                      jnp.float32)
                    relu = jnp.maximum(eh, 0.0)
                    contrib = relu * (diff1 + diff2)
                    cnt = Pd + Qd
                    acc = acc + jnp.where(cnt > 0.0, contrib, zeros)
                    cnv = cnv + jnp.take_along_axis(iq, lane15, axis=0)
                    cpv = cpv + jnp.take_along_axis(ip, lane15, axis=0)
                    return acc, cnv, cpv

                def scan_body(jj, carry):
                    acc, cnv, cpv = carry
                    acc, cnv, cpv = scan_step(jj * 2, acc, cnv, cpv)
                    acc, cnv, cpv = scan_step(jj * 2 + 1, acc, cnv, cpv)
                    return (acc, cnv, cpv)

                acc, _, _ = lax.fori_loop(
                    0, NUP // 2, scan_body, (zeros, zeros, zeros))
                loss = jnp.sum(acc) * 0.5
                obuf[...] = jnp.full((L,), loss)
                pltpu.sync_copy(obuf, out_hbm.at[img])

    return kern


_kern = _make_kernel()


def kernel(logits, labels):
    lf = logits.reshape(B * N)
    yf = labels.reshape(B * N)
    out = _kern(lf, yf)
    return jnp.mean(out[:, 0])


# speedup vs baseline: 1.8209x; 1.7897x over previous
import functools

import jax
import jax.numpy as jnp
from jax import lax
from jax.experimental import pallas as pl
from jax.experimental.pallas import tpu as pltpu
from jax.experimental.pallas import tpu_sc as plsc

B = 16
N = 384 * 384
BITS = 15
NB = 1 << BITS
SHIFT = 32 - BITS
HALF = 1 << (SHIFT - 1)
L = 16
CHUNK = 4096
NCHUNK = N // CHUNK
HCH = NCHUNK // 2
HIST = 2 * NB
NUP = (NB // 2) // L
UNROLL = 8
ZCH = HIST // L // 16


def _keys_chunk(lb, yb, ibuf, base_idx, ys):
    def vb(i, ysacc):
        for u in range(UNROLL):
            off = (i * UNROLL + u) * L
            l = lb[pl.ds(off, L)]
            y = yb[pl.ds(off, L)]
            yf = y.astype(jnp.float32)
            e = (1.0 + l) - 2.0 * (l * yf)
            bi = plsc.bitcast(e, jnp.int32)
            m = plsc.bitcast(bi >> 31, jnp.uint32) | jnp.uint32(0x80000000)
            key = plsc.bitcast(bi, jnp.uint32) ^ m
            bkt = (key + jnp.uint32(HALF)) >> SHIFT
            yu = plsc.bitcast(y, jnp.uint32)
            idx = plsc.bitcast(bkt | (yu << BITS), jnp.int32) + base_idx
            ibuf[pl.ds(off, L)] = idx
            ysacc = ysacc + yf
        return ysacc

    return lax.fori_loop(0, CHUNK // L // UNROLL, vb, ys)


def _make_kernel():
    mesh = plsc.VectorSubcoreMesh(
        core_axis_name="c", subcore_axis_name="s", num_cores=2,
        num_subcores=16)

    @functools.partial(
        pl.kernel,
        out_type=jax.ShapeDtypeStruct((B, L), jnp.float32),
        mesh=mesh,
        scratch_types=[
            pltpu.VMEM((CHUNK,), jnp.float32),
            pltpu.VMEM((CHUNK,), jnp.float32),
            pltpu.VMEM((CHUNK,), jnp.int32),
            pltpu.VMEM((CHUNK,), jnp.int32),
            pltpu.VMEM((CHUNK,), jnp.int32),
            pltpu.VMEM((CHUNK,), jnp.int32),
            pltpu.VMEM((CHUNK,), jnp.float32),
            pltpu.VMEM((HIST,), jnp.float32),
            pltpu.VMEM((L,), jnp.float32),
            pltpu.VMEM((L,), jnp.float32),
            pltpu.VMEM_SHARED((8 * HIST,), jnp.float32),
            pltpu.VMEM_SHARED((16, L), jnp.float32),
            pltpu.SemaphoreType.DMA,
            pltpu.SemaphoreType.DMA,
            pltpu.SemaphoreType.DMA,
            pltpu.SemaphoreType.DMA,
            pltpu.SemaphoreType.DMA,
            pltpu.SemaphoreType.DMA,
            pltpu.SemaphoreType.DMA,
        ],
        compiler_params=pltpu.CompilerParams(needs_layout_passes=False),
    )
    def kern(logits_hbm, labels_hbm, out_hbm, lbuf0, lbuf1, ybuf0, ybuf1,
             ibuf0, ibuf1, ones_c, hist, ysp, obuf, sh_hist, sh_ys,
             sl0, sy0, sl1, sy1, ss0, ss1, sz):
        c = lax.axis_index("c")
        s = lax.axis_index("s")
        slot = s % 8
        role = s // 8
        img = c * 8 + slot
        first = role * HCH
        base_idx = slot * HIST

        zeros = jnp.zeros((L,), jnp.float32)

        def start(k, lb, yb, sl, sy):
            off = img * N + k * CHUNK
            pltpu.async_copy(logits_hbm.at[pl.ds(off, CHUNK)], lb, sl)
            pltpu.async_copy(labels_hbm.at[pl.ds(off, CHUNK)], yb, sy)

        def wait(lb, yb, sl, sy, k):
            off = img * N + k * CHUNK
            pltpu.make_async_copy(
                logits_hbm.at[pl.ds(off, CHUNK)], lb, sl).wait()
            pltpu.make_async_copy(
                labels_hbm.at[pl.ds(off, CHUNK)], yb, sy).wait()

        start(first, lbuf0, ybuf0, sl0, sy0)

        with jax.named_scope("ph_zero"):
            ones = jnp.ones((L,), jnp.float32)

            def zero_body(i, _):
                for u in range(16):
                    hist[pl.ds((i * 16 + u) * L, L)] = zeros
                return 0

            lax.fori_loop(0, ZCH, zero_body, 0)

            def ones_body(i, _):
                for u in range(8):
                    ones_c[pl.ds((i * 8 + u) * L, L)] = ones
                return 0

            lax.fori_loop(0, CHUNK // L // 8, ones_body, 0)

            share = 8 * HIST // 16

            def zdma(i, _):
                pltpu.async_copy(
                    hist.at[pl.ds(0, 2048)],
                    sh_hist.at[pl.ds(s * share + i * 2048, 2048)], sz).wait()
                return 0

            lax.fori_loop(0, share // 2048, zdma, 0)

        plsc.subcore_barrier()

        with jax.named_scope("ph_hist"):
            def scatter(ib, ss):
                pltpu.async_copy(ones_c, sh_hist.at[ib], ss, add=True)

            def scatter_wait(ib, ss):
                pltpu.make_async_copy(ones_c, sh_hist.at[ib], ss).wait()

            def body(jj, ys):
                k = first + jj * 2
                start(k + 1, lbuf1, ybuf1, sl1, sy1)
                wait(lbuf0, ybuf0, sl0, sy0, k)

                @pl.when(jj > 0)
                def _():
                    scatter_wait(ibuf0, ss0)

                ys = _keys_chunk(lbuf0, ybuf0, ibuf0, base_idx, ys)
                scatter(ibuf0, ss0)

                @pl.when(jj < HCH // 2 - 1)
                def _():
                    start(k + 2, lbuf0, ybuf0, sl0, sy0)

                wait(lbuf1, ybuf1, sl1, sy1, k + 1)

                @pl.when(jj > 0)
                def _():
                    scatter_wait(ibuf1, ss1)

                ys = _keys_chunk(lbuf1, ybuf1, ibuf1, base_idx, ys)
                scatter(ibuf1, ss1)
                return ys

            ys = lax.fori_loop(0, HCH // 2, body, zeros)
            scatter_wait(ibuf0, ss0)
            scatter_wait(ibuf1, ss1)

        with jax.named_scope("ph_sync"):
            obuf[...] = ys
            pltpu.sync_copy(obuf, sh_ys.at[s])
            plsc.subcore_barrier()

        @pl.when(role == 0)
        def _():
            with jax.named_scope("ph_scan"):
                pltpu.sync_copy(sh_hist.at[pl.ds(base_idx, HIST)], hist)
                pltpu.sync_copy(sh_ys.at[s + 8], ysp)
                G = jnp.sum(ys) + jnp.sum(ysp[...])
                G2 = jnp.float32(N) - G

                iota = lax.iota(jnp.int32, L)
                one = jnp.full((L,), 1.0, jnp.float32)
                lane15 = jnp.full((L,), L - 1, jnp.int32)

                def scan_step(j, acc, cnv, cpv):
                    base = NB - (j + 1) * L
                    Pd = lax.rev(hist[pl.ds(NB + base, L)], (0,))
                    Qd = lax.rev(hist[pl.ds(base, L)], (0,))
                    ip = plsc.cumsum(Pd)
                    iq = plsc.cumsum(Qd)
                    pbar = cpv + (ip - Pd)
                    nbar = cnv + (iq - Qd)
                    d1 = G + nbar
                    num1 = (G - pbar) * Qd + Pd * d1
                    diff1 = jnp.where(d1 == 0.0, one,
                                      num1 / (d1 * (d1 + Qd)))
                    d2 = G2 + pbar
                    num2 = (G2 - nbar) * Pd + Qd * d2
                    diff2 = jnp.where(d2 == 0.0, one,
                                      num2 / (d2 * (d2 + Pd)))
                    hvec = (NB - 1 - j * L) - iota
                    center = plsc.bitcast(hvec, jnp.uint32) << SHIFT
                    eh = plsc.bitcast(center ^ jnp.uint32(0x80000000),
                                      jnp.float32)
                    relu = jnp.maximum(eh, 0.0)
                    contrib = relu * (diff1 + diff2)
                    cnt = Pd + Qd
                    acc = acc + jnp.where(cnt > 0.0, contrib, zeros)
                    cnv = cnv + jnp.take_along_axis(iq, lane15, axis=0)
                    cpv = cpv + jnp.take_along_axis(ip, lane15, axis=0)
                    return acc, cnv, cpv

                def scan_body(jj, carry):
                    acc, cnv, cpv = carry
                    acc, cnv, cpv = scan_step(jj * 2, acc, cnv, cpv)
                    acc, cnv, cpv = scan_step(jj * 2 + 1, acc, cnv, cpv)
                    return (acc, cnv, cpv)

                acc, _, _ = lax.fori_loop(
                    0, NUP // 2, scan_body, (zeros, zeros, zeros))
                loss = jnp.sum(acc) * 0.5
                obuf[...] = jnp.full((L,), loss)
                pltpu.sync_copy(obuf, out_hbm.at[img])

    return kern


_kern = _make_kernel()


def kernel(logits, labels):
    lf = logits.reshape(B * N)
    yf = labels.reshape(B * N)
    out = _kern(lf, yf)
    return jnp.mean(out[:, 0])


# speedup vs baseline: 1.8893x; 1.0375x over previous
import functools

import jax
import jax.numpy as jnp
from jax import lax
from jax.experimental import pallas as pl
from jax.experimental.pallas import tpu as pltpu
from jax.experimental.pallas import tpu_sc as plsc

B = 16
N = 384 * 384
BITS = 15
NB = 1 << BITS
SHIFT = 32 - BITS
HALF = 1 << (SHIFT - 1)
L = 16
CHUNK = 4096
NCHUNK = N // CHUNK
HCH = NCHUNK // 2
HIST = 2 * NB
NUP = (NB // 2) // L
UNROLL = 8
ZCH = HIST // L // 16


def _keys_chunk(lb, yb, ibuf, base_idx, ys):
    def vb(i, ysacc):
        for u in range(UNROLL):
            off = (i * UNROLL + u) * L
            l = lb[pl.ds(off, L)]
            y = yb[pl.ds(off, L)]
            yf = y.astype(jnp.float32)
            e = (1.0 + l) - 2.0 * (l * yf)
            bi = plsc.bitcast(e, jnp.int32)
            m = plsc.bitcast(bi >> 31, jnp.uint32) | jnp.uint32(0x80000000)
            key = plsc.bitcast(bi, jnp.uint32) ^ m
            bkt = (key + jnp.uint32(HALF)) >> SHIFT
            yu = plsc.bitcast(y, jnp.uint32)
            idx = plsc.bitcast(bkt | (yu << BITS), jnp.int32) + base_idx
            ibuf[pl.ds(off, L)] = idx
            ysacc = ysacc + yf
        return ysacc

    return lax.fori_loop(0, CHUNK // L // UNROLL, vb, ys)


def _make_kernel():
    mesh = plsc.VectorSubcoreMesh(
        core_axis_name="c", subcore_axis_name="s", num_cores=2,
        num_subcores=16)

    @functools.partial(
        pl.kernel,
        out_type=jax.ShapeDtypeStruct((B, L), jnp.float32),
        mesh=mesh,
        scratch_types=[
            pltpu.VMEM((CHUNK,), jnp.float32),
            pltpu.VMEM((CHUNK,), jnp.float32),
            pltpu.VMEM((CHUNK,), jnp.int32),
            pltpu.VMEM((CHUNK,), jnp.int32),
            pltpu.VMEM((CHUNK,), jnp.int32),
            pltpu.VMEM((CHUNK,), jnp.int32),
            pltpu.VMEM((CHUNK,), jnp.float32),
            pltpu.VMEM((HIST,), jnp.float32),
            pltpu.VMEM((L,), jnp.float32),
            pltpu.VMEM((L,), jnp.float32),
            pltpu.VMEM_SHARED((8 * HIST,), jnp.float32),
            pltpu.VMEM_SHARED((16, L), jnp.float32),
            pltpu.SemaphoreType.DMA,
            pltpu.SemaphoreType.DMA,
            pltpu.SemaphoreType.DMA,
            pltpu.SemaphoreType.DMA,
            pltpu.SemaphoreType.DMA,
            pltpu.SemaphoreType.DMA,
            pltpu.SemaphoreType.DMA,
        ],
        compiler_params=pltpu.CompilerParams(needs_layout_passes=False),
    )
    def kern(logits_hbm, labels_hbm, out_hbm, lbuf0, lbuf1, ybuf0, ybuf1,
             ibuf0, ibuf1, ones_c, hist, ysp, obuf, sh_hist, sh_ys,
             sl0, sy0, sl1, sy1, ss0, ss1, sz):
        c = lax.axis_index("c")
        s = lax.axis_index("s")
        slot = s % 8
        role = s // 8
        img = c * 8 + slot
        first = role * HCH
        base_idx = slot * HIST

        zeros = jnp.zeros((L,), jnp.float32)

        def start(k, lb, yb, sl, sy):
            off = img * N + k * CHUNK
            pltpu.async_copy(logits_hbm.at[pl.ds(off, CHUNK)], lb, sl)
            pltpu.async_copy(labels_hbm.at[pl.ds(off, CHUNK)], yb, sy)

        def wait(lb, yb, sl, sy, k):
            off = img * N + k * CHUNK
            pltpu.make_async_copy(
                logits_hbm.at[pl.ds(off, CHUNK)], lb, sl).wait()
            pltpu.make_async_copy(
                labels_hbm.at[pl.ds(off, CHUNK)], yb, sy).wait()

        start(first, lbuf0, ybuf0, sl0, sy0)

        with jax.named_scope("ph_zero"):
            ones = jnp.ones((L,), jnp.float32)

            def zero_body(i, _):
                for u in range(16):
                    hist[pl.ds((i * 16 + u) * L, L)] = zeros
                return 0

            lax.fori_loop(0, 8192 // L // 16, zero_body, 0)

            share = 8 * HIST // 16

            for i in range(4):
                pltpu.async_copy(
                    hist.at[pl.ds(0, 8192)],
                    sh_hist.at[pl.ds(s * share + i * 8192, 8192)], sz)

            def ones_body(i, _):
                for u in range(8):
                    ones_c[pl.ds((i * 8 + u) * L, L)] = ones
                return 0

            lax.fori_loop(0, CHUNK // L // 8, ones_body, 0)

            for i in range(4):
                pltpu.make_async_copy(
                    hist.at[pl.ds(0, 8192)],
                    sh_hist.at[pl.ds(s * share + i * 8192, 8192)], sz).wait()

        plsc.subcore_barrier()

        with jax.named_scope("ph_hist"):
            def scatter(ib, ss):
                pltpu.async_copy(ones_c, sh_hist.at[ib], ss, add=True)

            def scatter_wait(ib, ss):
                pltpu.make_async_copy(ones_c, sh_hist.at[ib], ss).wait()

            def body(jj, ys):
                k = first + jj * 2
                start(k + 1, lbuf1, ybuf1, sl1, sy1)
                wait(lbuf0, ybuf0, sl0, sy0, k)

                @pl.when(jj > 0)
                def _():
                    scatter_wait(ibuf0, ss0)

                ys = _keys_chunk(lbuf0, ybuf0, ibuf0, base_idx, ys)
                scatter(ibuf0, ss0)

                @pl.when(jj < HCH // 2 - 1)
                def _():
                    start(k + 2, lbuf0, ybuf0, sl0, sy0)

                wait(lbuf1, ybuf1, sl1, sy1, k + 1)

                @pl.when(jj > 0)
                def _():
                    scatter_wait(ibuf1, ss1)

                ys = _keys_chunk(lbuf1, ybuf1, ibuf1, base_idx, ys)
                scatter(ibuf1, ss1)
                return ys

            ys = lax.fori_loop(0, HCH // 2, body, zeros)
            scatter_wait(ibuf0, ss0)
            scatter_wait(ibuf1, ss1)

        with jax.named_scope("ph_sync"):
            obuf[...] = ys
            pltpu.sync_copy(obuf, sh_ys.at[s])
            plsc.subcore_barrier()

        @pl.when(role == 0)
        def _():
            with jax.named_scope("ph_scan"):
                pltpu.sync_copy(sh_hist.at[pl.ds(base_idx, HIST)], hist)
                pltpu.sync_copy(sh_ys.at[s + 8], ysp)
                G = jnp.sum(ys) + jnp.sum(ysp[...])
                G2 = jnp.float32(N) - G

                iota = lax.iota(jnp.int32, L)
                one = jnp.full((L,), 1.0, jnp.float32)
                lane15 = jnp.full((L,), L - 1, jnp.int32)

                def scan_step(j, acc, cnv, cpv):
                    base = NB - (j + 1) * L
                    Pd = lax.rev(hist[pl.ds(NB + base, L)], (0,))
                    Qd = lax.rev(hist[pl.ds(base, L)], (0,))
                    ip = plsc.cumsum(Pd)
                    iq = plsc.cumsum(Qd)
                    pbar = cpv + (ip - Pd)
                    nbar = cnv + (iq - Qd)
                    d1 = G + nbar
                    num1 = (G - pbar) * Qd + Pd * d1
                    diff1 = jnp.where(d1 == 0.0, one,
                                      num1 / (d1 * (d1 + Qd)))
                    d2 = G2 + pbar
                    num2 = (G2 - nbar) * Pd + Qd * d2
                    diff2 = jnp.where(d2 == 0.0, one,
                                      num2 / (d2 * (d2 + Pd)))
                    hvec = (NB - 1 - j * L) - iota
                    center = plsc.bitcast(hvec, jnp.uint32) << SHIFT
                    eh = plsc.bitcast(center ^ jnp.uint32(0x80000000),
                                      jnp.float32)
                    relu = jnp.maximum(eh, 0.0)
                    contrib = relu * (diff1 + diff2)
                    cnt = Pd + Qd
                    acc = acc + jnp.where(cnt > 0.0, contrib, zeros)
                    cnv = cnv + jnp.take_along_axis(iq, lane15, axis=0)
                    cpv = cpv + jnp.take_along_axis(ip, lane15, axis=0)
                    return acc, cnv, cpv

                def scan_body(jj, carry):
                    acc, cnv, cpv = carry
                    for u in range(4):
                        acc, cnv, cpv = scan_step(jj * 4 + u, acc, cnv, cpv)
                    return (acc, cnv, cpv)

                acc, _, _ = lax.fori_loop(
                    0, NUP // 4, scan_body, (zeros, zeros, zeros))
                loss = jnp.sum(acc) * 0.5
                obuf[...] = jnp.full((L,), loss)
                pltpu.sync_copy(obuf, out_hbm.at[img])

    return kern


_kern = _make_kernel()


def kernel(logits, labels):
    lf = logits.reshape(B * N)
    yf = labels.reshape(B * N)
    out = _kern(lf, yf)
    return jnp.mean(out[:, 0])
